# baseline (speedup 1.0000x reference)
import functools

import jax
import jax.numpy as jnp
from jax import lax
from jax.experimental import pallas as pl
from jax.experimental.pallas import tpu as pltpu
from jax.experimental.pallas import tpu_sc as plsc

B = 16384
D = 32
N = 1000000
L = 16
NC = 2
NS = 16
NW = NC * NS
BPW = B // NW
CHUNK = 128
NCHUNK = BPW // CHUNK
GROUPS = BPW // L

_mesh = plsc.VectorSubcoreMesh(core_axis_name="c", subcore_axis_name="s")


@functools.partial(
    pl.kernel,
    mesh=_mesh,
    out_type=jax.ShapeDtypeStruct((B,), jnp.float32),
    compiler_params=pltpu.CompilerParams(
        needs_layout_passes=False, use_tc_tiling_on_sc=False),
    scratch_types=[
        pltpu.VMEM((NCHUNK, CHUNK), jnp.int32),
        pltpu.VMEM((NCHUNK, CHUNK), jnp.int32),
        pltpu.VMEM((D, NCHUNK, CHUNK), jnp.int32),
        pltpu.VMEM((D, NCHUNK, CHUNK), jnp.int32),
        pltpu.VMEM((D * BPW,), jnp.float32),
        pltpu.VMEM((D * BPW,), jnp.float32),
        pltpu.VMEM((BPW,), jnp.float32),
        pltpu.SemaphoreType.DMA,
    ],
)
def _mf_sc(u_idx_hbm, i_idx_hbm, u_flat, i_flat, out_hbm,
           u_idx_v, i_idx_v, u_addr, i_addr, u_vals, i_vals, out_v, sem):
    wid = lax.axis_index("s") * NC + lax.axis_index("c")
    base = wid * BPW

    pltpu.sync_copy(u_idx_hbm.at[wid], u_idx_v)
    pltpu.sync_copy(i_idx_hbm.at[wid], i_idx_v)

    def build(d, carry):
        off = d * N
        for j in range(NCHUNK):
            for c in range(CHUNK // L):
                sl = pl.ds(c * L, L)
                u_addr[d, j, sl] = u_idx_v[j, sl] + off
                i_addr[d, j, sl] = i_idx_v[j, sl] + off
        return carry

    lax.fori_loop(0, D, build, 0)

    def fire(d, carry):
        for j in range(NCHUNK):
            dst = pl.ds(d * BPW + j * CHUNK, CHUNK)
            pltpu.async_copy(u_flat.at[u_addr.at[d, j]], u_vals.at[dst], sem)
            pltpu.async_copy(i_flat.at[i_addr.at[d, j]], i_vals.at[dst], sem)
        return carry

    lax.fori_loop(0, D, fire, 0)

    pltpu.make_async_copy(u_flat.at[pl.ds(0, D * BPW)], u_vals, sem).wait()
    pltpu.make_async_copy(i_flat.at[pl.ds(0, D * BPW)], i_vals, sem).wait()

    def body(g, carry):
        p0 = g * L
        acc = jnp.zeros((L,), jnp.float32)
        for d in range(D):
            uu = u_vals[pl.ds(d * BPW + p0, L)]
            ii = i_vals[pl.ds(d * BPW + p0, L)]
            acc = acc + uu * ii
        out_v[pl.ds(p0, L)] = 1.0 / (1.0 + jnp.exp(-acc))
        return carry

    lax.fori_loop(0, GROUPS, body, 0)

    pltpu.sync_copy(out_v, out_hbm.at[pl.ds(base, BPW)])


def kernel(user_indices, item_indices, user_table, item_table):
    u_idx = user_indices.astype(jnp.int32).reshape(NW, NCHUNK, CHUNK)
    i_idx = item_indices.astype(jnp.int32).reshape(NW, NCHUNK, CHUNK)
    u_flat = user_table.T.reshape(-1)
    i_flat = item_table.T.reshape(-1)
    return _mf_sc(u_idx, i_idx, u_flat, i_flat)


# speedup vs baseline: 5.7455x; 5.7455x over previous
import functools

import jax
import jax.numpy as jnp
from jax import lax
from jax.experimental import pallas as pl
from jax.experimental.pallas import tpu as pltpu
from jax.experimental.pallas import tpu_sc as plsc

B = 16384
D = 32
L = 16
NC = 2
NS = 16
NW = NC * NS
BPW = B // NW
CHUNK = 128
NCHUNK = BPW // CHUNK
GROUPS = BPW // L

_mesh = plsc.VectorSubcoreMesh(core_axis_name="c", subcore_axis_name="s")


@functools.partial(
    pl.kernel,
    mesh=_mesh,
    out_type=jax.ShapeDtypeStruct((B,), jnp.float32),
    compiler_params=pltpu.CompilerParams(
        needs_layout_passes=False, use_tc_tiling_on_sc=False),
    scratch_types=[
        pltpu.VMEM((NCHUNK, CHUNK), jnp.int32),
        pltpu.VMEM((NCHUNK, CHUNK), jnp.int32),
        pltpu.VMEM((BPW, D), jnp.float32),
        pltpu.VMEM((BPW, D), jnp.float32),
        pltpu.VMEM((BPW,), jnp.float32),
        pltpu.SemaphoreType.DMA,
    ],
)
def _mf_sc(u_idx_hbm, i_idx_hbm, u_tab_hbm, i_tab_hbm, out_hbm,
           u_idx_v, i_idx_v, u_rows, i_rows, out_v, sem):
    wid = lax.axis_index("s") * NC + lax.axis_index("c")
    base = wid * BPW

    pltpu.sync_copy(u_idx_hbm.at[wid], u_idx_v)
    pltpu.sync_copy(i_idx_hbm.at[wid], i_idx_v)

    copies = []
    for j in range(NCHUNK):
        copies.append(pltpu.async_copy(
            u_tab_hbm.at[u_idx_v.at[j]], u_rows.at[pl.ds(j * CHUNK, CHUNK)], sem))
        copies.append(pltpu.async_copy(
            i_tab_hbm.at[i_idx_v.at[j]], i_rows.at[pl.ds(j * CHUNK, CHUNK)], sem))
    for cp in copies:
        cp.wait()

    lane = lax.iota(jnp.int32, L)

    def body(g, carry):
        acc = jnp.zeros((L,), jnp.float32)
        for k in range(L):
            p = g * L + k
            u0 = u_rows[p, pl.ds(0, L)]
            u1 = u_rows[p, pl.ds(L, L)]
            i0 = i_rows[p, pl.ds(0, L)]
            i1 = i_rows[p, pl.ds(L, L)]
            s = jnp.sum(u0 * i0 + u1 * i1)
            acc = jnp.where(lane == k, s, acc)
        out_v[pl.ds(g * L, L)] = 1.0 / (1.0 + jnp.exp(-acc))
        return carry

    lax.fori_loop(0, GROUPS, body, 0)

    pltpu.sync_copy(out_v, out_hbm.at[pl.ds(base, BPW)])


def kernel(user_indices, item_indices, user_table, item_table):
    u_idx = user_indices.astype(jnp.int32).reshape(NW, NCHUNK, CHUNK)
    i_idx = item_indices.astype(jnp.int32).reshape(NW, NCHUNK, CHUNK)
    return _mf_sc(u_idx, i_idx, user_table, item_table)
